# baseline (speedup 1.0000x reference)
import functools

import jax
import jax.numpy as jnp
from jax import lax
from jax.experimental import pallas as pl
from jax.experimental.pallas import tpu as pltpu
from jax.experimental.pallas import tpu_sc as plsc

N = 10000
EDGES = 320000
F = 16
FP = 128
PACK = FP // F
NC = 2
NS = 16
NW = NC * NS
CHUNK = 128
SUPER = 16
NSUPER = 5
NBUFG = 4
NBUFS = 2
EPW = CHUNK * SUPER * NSUPER
EPAD = EPW * NW
EPR = EPAD // PACK
RPW = EPW // PACK
RPS = CHUNK * SUPER // PACK
NCH = EPW // CHUNK
NPT = 640
NPAD = NPT * NS
HALF = NPAD // NC
DUMP = HALF
ACC_H = 5248
ACC_T = ACC_H // NS
NPS = HALF // NS
EPS = EPAD // NS
NCH2 = EPS // CHUNK
NSUPER2 = NCH2 // SUPER


@functools.cache
def _mesh():
    return plsc.VectorSubcoreMesh(
        core_axis_name="c", subcore_axis_name="s", num_cores=NC, num_subcores=NS)


def _gather_body(table, idx, zeros, out, idx_v, rows_v, pack_v, sem, tbl_s):
    c = lax.axis_index("c")
    s = lax.axis_index("s")
    base = c * HALF
    pltpu.sync_copy(table.at[pl.ds(base + s * NPS, NPS)], tbl_s.at[pl.ds(s * NPS, NPS)])
    pltpu.sync_copy(zeros.at[pl.ds(0, ACC_H - HALF)], tbl_s.at[pl.ds(HALF, ACC_H - HALF)])
    pltpu.sync_copy(idx.at[s], idx_v)

    def rebase(r, carry):
        for gg in range(CHUNK // F):
            v = idx_v[r, pl.ds(gg * F, F)]
            t = v - base
            ok = (t >= 0) & (t < HALF)
            idx_v[r, pl.ds(gg * F, F)] = jnp.where(ok, t, DUMP)
        return carry

    lax.fori_loop(0, NCH2, rebase, 0)
    plsc.subcore_barrier()

    pltpu.async_copy(tbl_s.at[idx_v.at[0]], rows_v.at[0], sem)
    pltpu.async_copy(tbl_s.at[idx_v.at[1]], rows_v.at[1], sem)

    def do_super(k, carry):
        for j in range(SUPER):
            jb = j % 2
            pltpu.make_async_copy(zeros.at[pl.ds(0, CHUNK)], rows_v.at[jb], sem).wait()

            for g in range(CHUNK // PACK):
                for p in range(PACK):
                    pack_v[j * F + g, pl.ds(p * F, F)] = (
                        rows_v[jb, g * PACK + p, pl.ds(0, F)])

            ch = k * SUPER + j

            @pl.when(ch + 2 < NCH2)
            def _(jb=jb, ch=ch):
                pltpu.async_copy(tbl_s.at[idx_v.at[ch + 2]], rows_v.at[jb], sem)

        pltpu.sync_copy(pack_v, out.at[c, pl.ds(s * (EPS // PACK) + k * RPS, RPS)])
        return carry

    lax.fori_loop(0, NSUPER2, do_super, 0)


@functools.cache
def _sc_gather():
    return pl.kernel(
        _gather_body,
        out_type=jax.ShapeDtypeStruct((NC, EPR, FP), jnp.float32),
        mesh=_mesh(),
        scratch_types=[
            pltpu.VMEM((NCH2, CHUNK), jnp.int32),
            pltpu.VMEM((2, CHUNK, FP), jnp.float32),
            pltpu.VMEM((RPS, FP), jnp.float32),
            pltpu.SemaphoreType.DMA,
            pltpu.VMEM_SHARED((ACC_H, FP), jnp.float32),
        ],
    )


def _scatter_body(msg, idx, zeros, out, idx_v, rows_v, pack_v, sem, acc):
    c = lax.axis_index("c")
    s = lax.axis_index("s")
    base = c * HALF
    pltpu.sync_copy(idx.at[s], idx_v)
    for j in range(NBUFS):
        pltpu.sync_copy(zeros.at[pl.ds(0, CHUNK)], rows_v.at[j])
    pltpu.sync_copy(zeros.at[pl.ds(0, ACC_T)], acc.at[pl.ds(s * ACC_T, ACC_T)])

    def rebase(r, carry):
        for gg in range(CHUNK // F):
            v = idx_v[r, pl.ds(gg * F, F)]
            t = v - base
            ok = (t >= 0) & (t < HALF)
            idx_v[r, pl.ds(gg * F, F)] = jnp.where(ok, t, DUMP)
        return carry

    lax.fori_loop(0, NCH2, rebase, 0)
    plsc.subcore_barrier()

    def do_super(k, carry):
        pltpu.sync_copy(msg.at[pl.ds(s * (EPS // PACK) + k * RPS, RPS)], pack_v)
        for b in range(SUPER // NBUFS):
            descs = []
            for j in range(NBUFS):

                def expand(g, c2, j=j):
                    for p in range(PACK):
                        rows_v[j, g * PACK + p, pl.ds(0, F)] = (
                            pack_v[(b * NBUFS + j) * F + g, pl.ds(p * F, F)])
                    return c2

                lax.fori_loop(0, CHUNK // PACK, expand, 0)
                descs.append(pltpu.async_copy(
                    rows_v.at[j], acc.at[idx_v.at[k * SUPER + b * NBUFS + j]],
                    sem, add=True))
            for d in descs:
                d.wait()
        return carry

    lax.fori_loop(0, NSUPER2, do_super, 0)
    plsc.subcore_barrier()
    pltpu.sync_copy(acc.at[pl.ds(s * NPS, NPS)], out.at[pl.ds(base + s * NPS, NPS)])


@functools.cache
def _sc_scatter():
    return pl.kernel(
        _scatter_body,
        out_type=jax.ShapeDtypeStruct((NPAD, FP), jnp.float32),
        mesh=_mesh(),
        scratch_types=[
            pltpu.VMEM((NCH2, CHUNK), jnp.int32),
            pltpu.VMEM((NBUFS, CHUNK, FP), jnp.float32),
            pltpu.VMEM((RPS, FP), jnp.float32),
            pltpu.SemaphoreType.DMA,
            pltpu.VMEM_SHARED((ACC_H, FP), jnp.float32),
        ],
    )


BER = 256
GE = EPR // BER

def _edge_body(ea, xj0, xj1, w1r, b1r, tm, w2r, b2h, out):
    f32 = jnp.float32
    xj = (xj0[0] + xj1[0]).astype(jnp.bfloat16)
    hr = jnp.maximum(
        jnp.dot(ea[...], w1r[...], preferred_element_type=f32) + b1r[...], 0.0)
    xt = jnp.dot(xj, tm[...], preferred_element_type=f32)
    u = (hr * xt).astype(jnp.bfloat16)
    out[...] = (jnp.dot(u, w2r[...], preferred_element_type=f32)
                + jnp.dot(xj, b2h[...], preferred_element_type=f32))


_tc_edge = pl.pallas_call(
    _edge_body,
    grid=(GE,),
    in_specs=[
        pl.BlockSpec((BER, FP), lambda i: (i, 0)),
        pl.BlockSpec((1, BER, FP), lambda i: (0, i, 0)),
        pl.BlockSpec((1, BER, FP), lambda i: (1, i, 0)),
        pl.BlockSpec((FP, PACK * 256), lambda i: (0, 0)),
        pl.BlockSpec((1, PACK * 256), lambda i: (0, 0)),
        pl.BlockSpec((FP, PACK * 256), lambda i: (0, 0)),
        pl.BlockSpec((PACK * 256, FP), lambda i: (0, 0)),
        pl.BlockSpec((FP, FP), lambda i: (0, 0)),
    ],
    out_specs=pl.BlockSpec((BER, FP), lambda i: (i, 0)),
    out_shape=jax.ShapeDtypeStruct((EPR, FP), jnp.float32),
)


BN = 1280
GN = NPAD // BN

def _node_body(agg, x, rt, bt, out, *, do_relu, out_dtype):
    v = (agg[...]
         + jnp.dot(x[...], rt[...], preferred_element_type=jnp.float32)
         + bt[...])
    if do_relu:
        v = jnp.maximum(v, 0.0)
    out[...] = v.astype(out_dtype)


def _tc_node(agg, x, rt, bt, do_relu, out_dtype):
    return pl.pallas_call(
        functools.partial(_node_body, do_relu=do_relu, out_dtype=out_dtype),
        grid=(GN,),
        in_specs=[
            pl.BlockSpec((BN, FP), lambda i: (i, 0)),
            pl.BlockSpec((BN, FP), lambda i: (i, 0)),
            pl.BlockSpec((FP, FP), lambda i: (0, 0)),
            pl.BlockSpec((1, FP), lambda i: (0, 0)),
        ],
        out_specs=pl.BlockSpec((BN, FP), lambda i: (i, 0)),
        out_shape=jax.ShapeDtypeStruct((NPAD, FP), out_dtype),
    )(agg, x, rt, bt)


def kernel(x, edge_index, edge_attr,
           W1_0, b1_0, W2_0, b2_0, root_0, bias_0,
           W1_1, b1_1, W2_1, b2_1, root_1, bias_1,
           W1_2, b1_2, W2_2, b2_2, root_2, bias_2):
    f32 = jnp.float32
    src = edge_index[0].astype(jnp.int32)
    dst = edge_index[1].astype(jnp.int32)
    src_p = jnp.concatenate(
        [src, jnp.zeros((EPAD - EDGES,), jnp.int32)]).reshape(NS, NCH2, CHUNK)
    dst_p = jnp.concatenate(
        [dst, jnp.full((EPAD - EDGES,), N, jnp.int32)]).reshape(NS, NCH2, CHUNK)
    ea_p = jnp.concatenate(
        [edge_attr.astype(f32), jnp.zeros((EPAD - EDGES, 4), f32)], axis=0)
    ea_pk = jnp.pad(ea_p.reshape(EPR, PACK, 4),
                    ((0, 0), (0, 0), (0, F - 4))).reshape(EPR, FP).astype(jnp.bfloat16)
    x_t = jnp.zeros((NPAD, FP), f32).at[:N, :F].set(x.astype(f32))
    zeros_t = jnp.zeros((NPAD, FP), f32)

    jj = jnp.arange(256)
    kk = jnp.arange(F)
    rm = (jj[None, :] // F == kk[:, None]).astype(f32)
    tm = (jj[None, :] % F == kk[:, None]).astype(f32)
    eye8 = jnp.eye(PACK, dtype=f32)

    def expand(W1, b1, W2, b2, root, bias, ci, co, roff):
        w1r = jnp.pad(jnp.dot(W1, rm), ((0, F - 4), (0, 0)))
        b1r = jnp.dot(b1[None, :], rm)
        w2r = jnp.zeros((F, F, F), f32).at[:, roff:roff + ci, :co].set(
            W2.reshape(F, ci, co)).reshape(F * F, F)
        b2h = jnp.zeros((F, F), f32).at[roff:roff + ci, :co].set(b2.reshape(ci, co))
        rooth = jnp.zeros((F, F), f32).at[roff:roff + ci, :co].set(root)
        biash = jnp.zeros((F,), f32).at[:co].set(bias)
        bf16 = jnp.bfloat16
        return (
            jnp.kron(eye8, w1r).astype(bf16),
            jnp.tile(b1r, (1, PACK)),
            jnp.kron(eye8, tm).astype(bf16),
            jnp.kron(eye8, w2r).astype(bf16),
            jnp.kron(eye8, b2h).astype(bf16),
            jnp.zeros((FP, FP), f32).at[:F, :F].set(rooth),
            jnp.zeros((1, FP), f32).at[0, :F].set(biash),
        )

    layers = [
        expand(W1_0, b1_0, W2_0, b2_0, root_0, bias_0, 6, 16, 4),
        expand(W1_1, b1_1, W2_1, b2_1, root_1, bias_1, 16, 16, 0),
        expand(W1_2, b1_2, W2_2, b2_2, root_2, bias_2, 16, 4, 0),
    ]

    cur = x_t
    for li, w in enumerate(layers):
        xj = _sc_gather()(cur, src_p, zeros_t)
        msg = _tc_edge(ea_pk, xj, xj, *w[:5])
        agg = _sc_scatter()(msg, dst_p, zeros_t)
        cur = _tc_node(agg, cur, w[5], w[6], do_relu=(li < 2),
                       out_dtype=f32)
    return cur[:N, :4]


# speedup vs baseline: 1.0202x; 1.0202x over previous
import functools

import jax
import jax.numpy as jnp
from jax import lax
from jax.experimental import pallas as pl
from jax.experimental.pallas import tpu as pltpu
from jax.experimental.pallas import tpu_sc as plsc

N = 10000
EDGES = 320000
F = 16
FP = 128
PACK = FP // F
NC = 2
NS = 16
NW = NC * NS
CHUNK = 128
SUPER = 16
NSUPER = 5
NBUFG = 4
NBUFS = 2
EPW = CHUNK * SUPER * NSUPER
EPAD = EPW * NW
EPR = EPAD // PACK
RPW = EPW // PACK
RPS = CHUNK * SUPER // PACK
NCH = EPW // CHUNK
NPT = 640
NPAD = NPT * NS
HALF = NPAD // NC
DUMP = HALF
ACC_H = 5248
ACC_T = ACC_H // NS
NPS = HALF // NS
EPS = EPAD // NS
NCH2 = EPS // CHUNK
NSUPER2 = NCH2 // SUPER


@functools.cache
def _mesh():
    return plsc.VectorSubcoreMesh(
        core_axis_name="c", subcore_axis_name="s", num_cores=NC, num_subcores=NS)


def _gather_body(table, idx, zeros, out, idx_v, rows_v, pack_v, sem, tbl_s):
    c = lax.axis_index("c")
    s = lax.axis_index("s")
    base = c * HALF
    pltpu.sync_copy(table.at[pl.ds(base + s * NPS, NPS)], tbl_s.at[pl.ds(s * NPS, NPS)])
    pltpu.sync_copy(zeros.at[pl.ds(0, ACC_H - HALF)], tbl_s.at[pl.ds(HALF, ACC_H - HALF)])
    pltpu.sync_copy(idx.at[s], idx_v)

    def rebase(r, carry):
        for gg in range(CHUNK // F):
            v = idx_v[r, pl.ds(gg * F, F)]
            t = v - base
            ok = (t >= 0) & (t < HALF)
            idx_v[r, pl.ds(gg * F, F)] = jnp.where(ok, t, DUMP)
        return carry

    lax.fori_loop(0, NCH2, rebase, 0)
    plsc.subcore_barrier()

    pltpu.async_copy(tbl_s.at[idx_v.at[0]], rows_v.at[0], sem)
    pltpu.async_copy(tbl_s.at[idx_v.at[1]], rows_v.at[1], sem)

    def chunk_pair(i, carry):
        for jb in range(2):
            ch = 2 * i + jb
            pltpu.make_async_copy(zeros.at[pl.ds(0, CHUNK)], rows_v.at[jb], sem).wait()

            def compact(g, c2, jb=jb):
                for p in range(PACK):
                    pack_v[(ch % SUPER) * F + g, pl.ds(p * F, F)] = (
                        rows_v[jb, g * PACK + p, pl.ds(0, F)])
                return c2

            lax.fori_loop(0, CHUNK // PACK, compact, 0)

            @pl.when(ch + 2 < NCH2)
            def _(jb=jb, ch=ch):
                pltpu.async_copy(tbl_s.at[idx_v.at[ch + 2]], rows_v.at[jb], sem)

            @pl.when(ch % SUPER == SUPER - 1)
            def _(ch=ch):
                pltpu.sync_copy(
                    pack_v,
                    out.at[c, pl.ds(s * (EPS // PACK) + (ch // SUPER) * RPS, RPS)])
        return carry

    lax.fori_loop(0, NCH2 // 2, chunk_pair, 0)


@functools.cache
def _sc_gather():
    return pl.kernel(
        _gather_body,
        out_type=jax.ShapeDtypeStruct((NC, EPR, FP), jnp.float32),
        mesh=_mesh(),
        scratch_types=[
            pltpu.VMEM((NCH2, CHUNK), jnp.int32),
            pltpu.VMEM((2, CHUNK, FP), jnp.float32),
            pltpu.VMEM((RPS, FP), jnp.float32),
            pltpu.SemaphoreType.DMA,
            pltpu.VMEM_SHARED((ACC_H, FP), jnp.float32),
        ],
    )


def _scatter_body(msg, idx, zeros, out, idx_v, rows_v, pack_v, sem, acc):
    c = lax.axis_index("c")
    s = lax.axis_index("s")
    base = c * HALF
    pltpu.sync_copy(idx.at[s], idx_v)
    for j in range(NBUFS):
        pltpu.sync_copy(zeros.at[pl.ds(0, CHUNK)], rows_v.at[j])
    pltpu.sync_copy(zeros.at[pl.ds(0, ACC_T)], acc.at[pl.ds(s * ACC_T, ACC_T)])

    def rebase(r, carry):
        for gg in range(CHUNK // F):
            v = idx_v[r, pl.ds(gg * F, F)]
            t = v - base
            ok = (t >= 0) & (t < HALF)
            idx_v[r, pl.ds(gg * F, F)] = jnp.where(ok, t, DUMP)
        return carry

    lax.fori_loop(0, NCH2, rebase, 0)
    plsc.subcore_barrier()

    def do_super(k, carry):
        pltpu.sync_copy(msg.at[pl.ds(s * (EPS // PACK) + k * RPS, RPS)], pack_v)
        for b in range(SUPER // NBUFS):
            descs = []
            for j in range(NBUFS):

                def expand(g, c2, j=j):
                    for p in range(PACK):
                        rows_v[j, g * PACK + p, pl.ds(0, F)] = (
                            pack_v[(b * NBUFS + j) * F + g, pl.ds(p * F, F)])
                    return c2

                lax.fori_loop(0, CHUNK // PACK, expand, 0)
                descs.append(pltpu.async_copy(
                    rows_v.at[j], acc.at[idx_v.at[k * SUPER + b * NBUFS + j]],
                    sem, add=True))
            for d in descs:
                d.wait()
        return carry

    lax.fori_loop(0, NSUPER2, do_super, 0)
    plsc.subcore_barrier()
    pltpu.sync_copy(acc.at[pl.ds(s * NPS, NPS)], out.at[pl.ds(base + s * NPS, NPS)])


@functools.cache
def _sc_scatter():
    return pl.kernel(
        _scatter_body,
        out_type=jax.ShapeDtypeStruct((NPAD, FP), jnp.float32),
        mesh=_mesh(),
        scratch_types=[
            pltpu.VMEM((NCH2, CHUNK), jnp.int32),
            pltpu.VMEM((NBUFS, CHUNK, FP), jnp.float32),
            pltpu.VMEM((RPS, FP), jnp.float32),
            pltpu.SemaphoreType.DMA,
            pltpu.VMEM_SHARED((ACC_H, FP), jnp.float32),
        ],
    )


BER = 256
GE = EPR // BER

def _edge_body(ea, xj0, xj1, w1r, b1r, tm, w2r, b2h, out):
    f32 = jnp.float32
    xj = (xj0[0] + xj1[0]).astype(jnp.bfloat16)
    hr = jnp.maximum(
        jnp.dot(ea[...], w1r[...], preferred_element_type=f32) + b1r[...], 0.0)
    xt = jnp.dot(xj, tm[...], preferred_element_type=f32)
    u = (hr * xt).astype(jnp.bfloat16)
    out[...] = (jnp.dot(u, w2r[...], preferred_element_type=f32)
                + jnp.dot(xj, b2h[...], preferred_element_type=f32))


_tc_edge = pl.pallas_call(
    _edge_body,
    grid=(GE,),
    in_specs=[
        pl.BlockSpec((BER, FP), lambda i: (i, 0)),
        pl.BlockSpec((1, BER, FP), lambda i: (0, i, 0)),
        pl.BlockSpec((1, BER, FP), lambda i: (1, i, 0)),
        pl.BlockSpec((FP, PACK * 256), lambda i: (0, 0)),
        pl.BlockSpec((1, PACK * 256), lambda i: (0, 0)),
        pl.BlockSpec((FP, PACK * 256), lambda i: (0, 0)),
        pl.BlockSpec((PACK * 256, FP), lambda i: (0, 0)),
        pl.BlockSpec((FP, FP), lambda i: (0, 0)),
    ],
    out_specs=pl.BlockSpec((BER, FP), lambda i: (i, 0)),
    out_shape=jax.ShapeDtypeStruct((EPR, FP), jnp.float32),
)


BN = 1280
GN = NPAD // BN

def _node_body(agg, x, rt, bt, out, *, do_relu, out_dtype):
    v = (agg[...]
         + jnp.dot(x[...], rt[...], preferred_element_type=jnp.float32)
         + bt[...])
    if do_relu:
        v = jnp.maximum(v, 0.0)
    out[...] = v.astype(out_dtype)


def _tc_node(agg, x, rt, bt, do_relu, out_dtype):
    return pl.pallas_call(
        functools.partial(_node_body, do_relu=do_relu, out_dtype=out_dtype),
        grid=(GN,),
        in_specs=[
            pl.BlockSpec((BN, FP), lambda i: (i, 0)),
            pl.BlockSpec((BN, FP), lambda i: (i, 0)),
            pl.BlockSpec((FP, FP), lambda i: (0, 0)),
            pl.BlockSpec((1, FP), lambda i: (0, 0)),
        ],
        out_specs=pl.BlockSpec((BN, FP), lambda i: (i, 0)),
        out_shape=jax.ShapeDtypeStruct((NPAD, FP), out_dtype),
    )(agg, x, rt, bt)


def kernel(x, edge_index, edge_attr,
           W1_0, b1_0, W2_0, b2_0, root_0, bias_0,
           W1_1, b1_1, W2_1, b2_1, root_1, bias_1,
           W1_2, b1_2, W2_2, b2_2, root_2, bias_2):
    f32 = jnp.float32
    src = edge_index[0].astype(jnp.int32)
    dst = edge_index[1].astype(jnp.int32)
    src_p = jnp.concatenate(
        [src, jnp.zeros((EPAD - EDGES,), jnp.int32)]).reshape(NS, NCH2, CHUNK)
    dst_p = jnp.concatenate(
        [dst, jnp.full((EPAD - EDGES,), N, jnp.int32)]).reshape(NS, NCH2, CHUNK)
    ea_p = jnp.concatenate(
        [edge_attr.astype(f32), jnp.zeros((EPAD - EDGES, 4), f32)], axis=0)
    ea_pk = jnp.pad(ea_p.reshape(EPR, PACK, 4),
                    ((0, 0), (0, 0), (0, F - 4))).reshape(EPR, FP).astype(jnp.bfloat16)
    x_t = jnp.zeros((NPAD, FP), f32).at[:N, :F].set(x.astype(f32))
    zeros_t = jnp.zeros((NPAD, FP), f32)

    jj = jnp.arange(256)
    kk = jnp.arange(F)
    rm = (jj[None, :] // F == kk[:, None]).astype(f32)
    tm = (jj[None, :] % F == kk[:, None]).astype(f32)
    eye8 = jnp.eye(PACK, dtype=f32)

    def expand(W1, b1, W2, b2, root, bias, ci, co, roff):
        w1r = jnp.pad(jnp.dot(W1, rm), ((0, F - 4), (0, 0)))
        b1r = jnp.dot(b1[None, :], rm)
        w2r = jnp.zeros((F, F, F), f32).at[:, roff:roff + ci, :co].set(
            W2.reshape(F, ci, co)).reshape(F * F, F)
        b2h = jnp.zeros((F, F), f32).at[roff:roff + ci, :co].set(b2.reshape(ci, co))
        rooth = jnp.zeros((F, F), f32).at[roff:roff + ci, :co].set(root)
        biash = jnp.zeros((F,), f32).at[:co].set(bias)
        bf16 = jnp.bfloat16
        return (
            jnp.kron(eye8, w1r).astype(bf16),
            jnp.tile(b1r, (1, PACK)),
            jnp.kron(eye8, tm).astype(bf16),
            jnp.kron(eye8, w2r).astype(bf16),
            jnp.kron(eye8, b2h).astype(bf16),
            jnp.zeros((FP, FP), f32).at[:F, :F].set(rooth),
            jnp.zeros((1, FP), f32).at[0, :F].set(biash),
        )

    layers = [
        expand(W1_0, b1_0, W2_0, b2_0, root_0, bias_0, 6, 16, 4),
        expand(W1_1, b1_1, W2_1, b2_1, root_1, bias_1, 16, 16, 0),
        expand(W1_2, b1_2, W2_2, b2_2, root_2, bias_2, 16, 4, 0),
    ]

    cur = x_t
    for li, w in enumerate(layers):
        xj = _sc_gather()(cur, src_p, zeros_t)
        msg = _tc_edge(ea_pk, xj, xj, *w[:5])
        agg = _sc_scatter()(msg, dst_p, zeros_t)
        cur = _tc_node(agg, cur, w[5], w[6], do_relu=(li < 2),
                       out_dtype=f32)
    return cur[:N, :4]
